# Initial kernel scaffold; baseline (speedup 1.0000x reference)
import jax
import jax.numpy as jnp
from jax.experimental import pallas as pl


def kernel(input_ids, params):
    raise NotImplementedError("write your pallas kernel here")


# speedup vs baseline: 1.4392x; 1.4392x over previous
import functools

import jax
import jax.numpy as jnp
import numpy as np
from jax import lax
from jax.experimental import pallas as pl
from jax.experimental.pallas import tpu as pltpu
from jax.experimental.pallas import tpu_sc as plsc

B = 1; S = 2048; V = 32000; D = 768; H = 12; DH = D // H
NL = 64; NS = 4; NR = 4; TOPK = 2; L = 4; DFF = 1536
T = B * S

TB = 1024
TQ = 512
FB = 768
VB = 1280

_SCALE = 1.0 / np.sqrt(DH)


def _ln(x, g, b):
    m = jnp.mean(x, axis=-1, keepdims=True)
    v = jnp.mean((x - m) ** 2, axis=-1, keepdims=True)
    return (x - m) / jnp.sqrt(v + 1e-5) * g + b


def _b16(a):
    return a.astype(jnp.bfloat16)


def _dot3(a, b):
    return jnp.dot(_b16(a), _b16(b), preferred_element_type=jnp.float32)


def _dot3_t(a, b):
    return lax.dot_general(_b16(a), _b16(b), (((1,), (1,)), ((), ())),
                           preferred_element_type=jnp.float32)


def _embed(table, ids):
    info = plsc.get_sparse_core_info()
    nw = info.num_cores * info.num_subcores
    bpw = T // nw
    mesh = plsc.VectorSubcoreMesh(core_axis_name="c", subcore_axis_name="s")

    @functools.partial(
        pl.kernel, mesh=mesh,
        out_type=jax.ShapeDtypeStruct((T, D), jnp.float32),
        scratch_types=[
            pltpu.VMEM((bpw,), jnp.int32),
            pltpu.VMEM((bpw, D), jnp.float32),
            pltpu.SemaphoreType.DMA,
        ],
    )
    def emb_k(table_hbm, idx_hbm, out_hbm, idx_v, rows_v, sem):
        wid = lax.axis_index("s") * info.num_cores + lax.axis_index("c")
        base = wid * bpw
        pltpu.sync_copy(idx_hbm.at[pl.ds(base, bpw)], idx_v)
        pltpu.async_copy(table_hbm.at[idx_v], rows_v, sem).wait()
        pltpu.sync_copy(rows_v, out_hbm.at[pl.ds(base, bpw)])

    return emb_k(table, ids)


def _qkv_body(x_ref, g_ref, b_ref, wq_ref, wdkv_ref, wuk_ref, wuv_ref,
              q_ref, k_ref, v_ref):
    h = _ln(x_ref[...], g_ref[0], b_ref[0])
    q_ref[...] = _dot3(h, wq_ref[0])
    c = _dot3(h, wdkv_ref[0])
    k_ref[...] = _dot3(c, wuk_ref[0])
    v_ref[...] = _dot3(c, wuv_ref[0])


def _qkv(x, p, i):
    nt = T // TB
    f32 = jnp.float32
    return pl.pallas_call(
        _qkv_body,
        grid=(nt,),
        in_specs=[
            pl.BlockSpec((TB, D), lambda t: (t, 0)),
            pl.BlockSpec((1, 1, D), lambda t: (i, 0, 0)),
            pl.BlockSpec((1, 1, D), lambda t: (i, 0, 0)),
            pl.BlockSpec((1, D, D), lambda t: (i, 0, 0)),
            pl.BlockSpec((1, D, NL), lambda t: (i, 0, 0)),
            pl.BlockSpec((1, NL, D), lambda t: (i, 0, 0)),
            pl.BlockSpec((1, NL, D), lambda t: (i, 0, 0)),
        ],
        out_specs=[
            pl.BlockSpec((TB, D), lambda t: (t, 0)),
            pl.BlockSpec((TB, D), lambda t: (t, 0)),
            pl.BlockSpec((TB, D), lambda t: (t, 0)),
        ],
        out_shape=[jax.ShapeDtypeStruct((T, D), f32)] * 3,
    )(x, p["ln1_g"].reshape(L, 1, D), p["ln1_b"].reshape(L, 1, D),
      p["Wq"], p["Wdkv"], p["Wuk"], p["Wuv"])


HPB = 2


CK = 1024


def _attn_body(q_ref, k_ref, v_ref, o_ref):
    iq = pl.program_id(1)
    q = q_ref[...]
    k = k_ref[...]
    v = v_ref[...]
    row = iq * TQ + lax.broadcasted_iota(jnp.int32, (TQ, CK), 0)
    col = lax.broadcasted_iota(jnp.int32, (TQ, CK), 1)
    outs = []
    for hh in range(HPB):
        qh = q[:, hh * DH:(hh + 1) * DH]
        M = jnp.full((TQ, 1), -jnp.inf, jnp.float32)
        N = jnp.zeros((TQ, DH), jnp.float32)
        Dn = jnp.zeros((TQ, 1), jnp.float32)
        for c0 in range(0, S, CK):
            kh = k[c0:c0 + CK, hh * DH:(hh + 1) * DH]
            vh = v[c0:c0 + CK, hh * DH:(hh + 1) * DH]
            s = _dot3_t(qh, kh) * _SCALE
            s = jnp.where(c0 + col <= row, s, jnp.float32(-1e9))
            mc = jnp.max(s, axis=-1, keepdims=True)
            Mn = jnp.maximum(M, mc)
            e = jnp.exp(s - Mn)
            alpha = jnp.exp(M - Mn)
            N = N * alpha + _dot3(e, vh)
            Dn = Dn * alpha + jnp.sum(e, axis=-1, keepdims=True)
            M = Mn
        outs.append(N / Dn)
    o_ref[...] = jnp.concatenate(outs, axis=1)


def _attn(q, k, v):
    nq = T // TQ
    hw = HPB * DH
    return pl.pallas_call(
        _attn_body,
        grid=(H // HPB, nq),
        in_specs=[
            pl.BlockSpec((TQ, hw), lambda h, t: (t, h)),
            pl.BlockSpec((S, hw), lambda h, t: (0, h)),
            pl.BlockSpec((S, hw), lambda h, t: (0, h)),
        ],
        out_specs=pl.BlockSpec((TQ, hw), lambda h, t: (t, h)),
        out_shape=jax.ShapeDtypeStruct((T, D), jnp.float32),
    )(q, k, v)


def _proj_gate_body(x_ref, o_ref, wo_ref, g_ref, b_ref, gw_ref,
                    xo_ref, h2_ref, w4_ref):
    xo = x_ref[...] + _dot3(o_ref[...], wo_ref[0])
    xo_ref[...] = xo
    h2 = _ln(xo, g_ref[0], b_ref[0])
    h2_ref[...] = h2
    gl = _dot3(h2, gw_ref[0])
    gl = gl - jnp.max(gl, axis=-1, keepdims=True)
    eg = jnp.exp(gl)
    gp = eg / jnp.sum(eg, axis=-1, keepdims=True)
    idx = lax.broadcasted_iota(jnp.int32, gp.shape, 1)
    m1 = jnp.max(gp, axis=-1, keepdims=True)
    i1 = jnp.min(jnp.where(gp == m1, idx, NR), axis=-1, keepdims=True)
    masked = jnp.where(idx == i1, jnp.float32(-1e30), gp)
    m2 = jnp.max(masked, axis=-1, keepdims=True)
    i2 = jnp.min(jnp.where(masked == m2, idx, NR), axis=-1, keepdims=True)
    sel = (idx == i1) | (idx == i2)
    w4_ref[...] = jnp.where(sel, gp, 0.0) / (m1 + m2)


def _proj_gate(x, o, p, i):
    nt = T // TB
    f32 = jnp.float32
    return pl.pallas_call(
        _proj_gate_body,
        grid=(nt,),
        in_specs=[
            pl.BlockSpec((TB, D), lambda t: (t, 0)),
            pl.BlockSpec((TB, D), lambda t: (t, 0)),
            pl.BlockSpec((1, D, D), lambda t: (i, 0, 0)),
            pl.BlockSpec((1, 1, D), lambda t: (i, 0, 0)),
            pl.BlockSpec((1, 1, D), lambda t: (i, 0, 0)),
            pl.BlockSpec((1, D, NR), lambda t: (i, 0, 0)),
        ],
        out_specs=[
            pl.BlockSpec((TB, D), lambda t: (t, 0)),
            pl.BlockSpec((TB, D), lambda t: (t, 0)),
            pl.BlockSpec((TB, NR), lambda t: (t, 0)),
        ],
        out_shape=[
            jax.ShapeDtypeStruct((T, D), f32),
            jax.ShapeDtypeStruct((T, D), f32),
            jax.ShapeDtypeStruct((T, NR), f32),
        ],
    )(x, o, p["Wo"], p["ln2_g"].reshape(L, 1, D), p["ln2_b"].reshape(L, 1, D),
      p["gate_w"])


def _moe_shared_body(h2_ref, w1_ref, w2_ref, xo_ref, y_ref):
    e = pl.program_id(0)
    f = pl.program_id(1)
    t1 = _dot3(h2_ref[...], w1_ref[0, 0])
    t1 = t1 * jax.nn.sigmoid(t1)
    part = _dot3(t1, w2_ref[0, 0])

    @pl.when((e == 0) & (f == 0))
    def _():
        y_ref[...] = xo_ref[...] + part

    @pl.when((e > 0) | (f > 0))
    def _():
        y_ref[...] += part


def _moe_routed_body(h2_ref, w1_ref, w2_ref, w4_ref, y0_ref, y_ref):
    e = pl.program_id(0)
    f = pl.program_id(1)
    t1 = _dot3(h2_ref[...], w1_ref[0, 0])
    t1 = t1 * jax.nn.sigmoid(t1)
    part = _dot3(t1, w2_ref[0, 0])
    idx = lax.broadcasted_iota(jnp.int32, (T, NR), 1)
    we = jnp.sum(jnp.where(idx == e, w4_ref[...], 0.0), axis=-1, keepdims=True)
    part = _b16(part).astype(jnp.float32) * _b16(we).astype(jnp.float32)

    @pl.when((e == 0) & (f == 0))
    def _():
        y_ref[...] = y0_ref[...] + part

    @pl.when((e > 0) | (f > 0))
    def _():
        y_ref[...] += part


def _moe(h2, xo, w4, p, i):
    nf = DFF // FB
    f32 = jnp.float32
    y1 = pl.pallas_call(
        _moe_shared_body,
        grid=(NS, nf),
        in_specs=[
            pl.BlockSpec((T, D), lambda e, f: (0, 0)),
            pl.BlockSpec((1, 1, D, FB), lambda e, f: (i, e, 0, f)),
            pl.BlockSpec((1, 1, FB, D), lambda e, f: (i, e, f, 0)),
            pl.BlockSpec((T, D), lambda e, f: (0, 0)),
        ],
        out_specs=pl.BlockSpec((T, D), lambda e, f: (0, 0)),
        out_shape=jax.ShapeDtypeStruct((T, D), f32),
    )(h2, p["ws1"], p["ws2"], xo)
    y = pl.pallas_call(
        _moe_routed_body,
        grid=(NR, nf),
        in_specs=[
            pl.BlockSpec((T, D), lambda e, f: (0, 0)),
            pl.BlockSpec((1, 1, D, FB), lambda e, f: (i, e, 0, f)),
            pl.BlockSpec((1, 1, FB, D), lambda e, f: (i, e, f, 0)),
            pl.BlockSpec((T, NR), lambda e, f: (0, 0)),
            pl.BlockSpec((T, D), lambda e, f: (0, 0)),
        ],
        out_specs=pl.BlockSpec((T, D), lambda e, f: (0, 0)),
        out_shape=jax.ShapeDtypeStruct((T, D), f32),
    )(h2, p["wr1"], p["wr2"], w4, y1)
    return y


def _head_body(x_ref, g_ref, b_ref, w_ref, hb_ref, out_ref):
    xf = _ln(x_ref[...], g_ref[...], b_ref[...])
    out_ref[...] = (_dot3(xf, w_ref[...])
                    + hb_ref[...])


def _head(x, p):
    nv = V // VB
    return pl.pallas_call(
        _head_body,
        grid=(nv,),
        in_specs=[
            pl.BlockSpec((T, D), lambda v: (0, 0)),
            pl.BlockSpec((1, D), lambda v: (0, 0)),
            pl.BlockSpec((1, D), lambda v: (0, 0)),
            pl.BlockSpec((D, VB), lambda v: (0, v)),
            pl.BlockSpec((1, VB), lambda v: (0, v)),
        ],
        out_specs=pl.BlockSpec((T, VB), lambda v: (0, v)),
        out_shape=jax.ShapeDtypeStruct((T, V), jnp.float32),
    )(x, p["fln_g"].reshape(1, D), p["fln_b"].reshape(1, D),
      p["head_w"], p["head_b"].reshape(1, V))


def kernel(input_ids, params):
    ids = input_ids.reshape(T)
    x = _embed(params["emb"], ids)
    for i in range(L):
        q, k, v = _qkv(x, params, i)
        o = _attn(q, k, v)
        x, h2, w4 = _proj_gate(x, o, params, i)
        x = _moe(h2, x, w4, params, i)
    logits = _head(x, params)
    return logits.reshape(B, S, V)
